# baseline (speedup 1.0000x reference)
import jax
import jax.numpy as jnp
from jax.experimental import pallas as pl
from jax.experimental.pallas import tpu as pltpu
from jax.experimental.pallas import tpu_sc as plsc

_N = 10000
_NPAD = 10240
_D = 128
_E = 320000
_OUT = 47
_K = 80
_IB = 16
_NSUB = 16
_NCORES = 2
_W = _NCORES * _NSUB
_SUPERS = _E // (_IB * _K)
_NITER = -(-_SUPERS // _W)
_STRIPE = _NPAD // _NSUB

_NB = 4
_GL = 3


def _agg_body(h_hbm, src_hbm, dst_hbm, out_hbm, acc, srcs, dsts, rows, sems):
    c = jax.lax.axis_index("c")
    s = jax.lax.axis_index("s")
    wid = s * _NCORES + c
    row0 = s * _STRIPE
    sem_i, gsems, ssems = sems

    def fire_idx(sc, bank):
        base = sc * (_IB * _K)
        pltpu.async_copy(
            src_hbm.at[pl.ds(base, _IB * _K)], srcs[bank], sem_i)
        pltpu.async_copy(
            dst_hbm.at[pl.ds(base, _IB * _K)], dsts[bank], sem_i)

    def drain_idx(sc, bank):
        base = sc * (_IB * _K)
        pltpu.make_async_copy(
            src_hbm.at[pl.ds(base, _IB * _K)], srcs[bank], sem_i).wait()
        pltpu.make_async_copy(
            dst_hbm.at[pl.ds(base, _IB * _K)], dsts[bank], sem_i).wait()

    fire_idx(wid, 0)
    pltpu.async_copy(
        h_hbm.at[pl.ds(row0, _STRIPE)], acc.at[pl.ds(row0, _STRIPE)], sem_i
    ).wait()
    plsc.subcore_barrier()

    def do_super(sc, bank):
        drain_idx(sc, bank)

        nsc = sc + _W

        @pl.when(nsc < _SUPERS)
        def _():
            fire_idx(nsc, bank ^ 1)

        gd = [None] * _IB
        sd = [None] * _NB
        for j in range(min(_GL, _IB)):
            gd[j] = pltpu.async_copy(
                h_hbm.at[srcs[bank].at[pl.ds(j * _K, _K)]],
                rows[j % _NB], gsems[j % _NB])
        for j in range(_IB):
            jj = j + _GL
            if jj < _IB:
                b2 = jj % _NB
                if sd[b2] is not None:
                    sd[b2].wait()
                    sd[b2] = None
                gd[jj] = pltpu.async_copy(
                    h_hbm.at[srcs[bank].at[pl.ds(jj * _K, _K)]],
                    rows[b2], gsems[b2])
            b = j % _NB
            gd[j].wait()
            sd[b] = pltpu.async_copy(
                rows[b], acc.at[dsts[bank].at[pl.ds(j * _K, _K)]],
                ssems[b], add=True)
        for d in sd:
            if d is not None:
                d.wait()

    @pl.loop(0, _NITER, step=2)
    def _(g):
        for k in range(2):
            sc = (g + k) * _W + wid

            @pl.when(sc < _SUPERS)
            def _(sc=sc, k=k):
                do_super(sc, k)

    plsc.subcore_barrier()
    pltpu.async_copy(
        acc.at[pl.ds(row0, _STRIPE)], out_hbm.at[c, pl.ds(row0, _STRIPE)], sem_i
    ).wait()


def _aggregate(h, src, dst):
    kern = pl.kernel(
        _agg_body,
        out_type=jax.ShapeDtypeStruct((_NCORES, _NPAD, _D), jnp.float32),
        mesh=plsc.VectorSubcoreMesh(core_axis_name="c", subcore_axis_name="s"),
        scratch_types=[
            pltpu.VMEM_SHARED((_NPAD, _D), jnp.float32),
            [pltpu.VMEM((_IB * _K,), jnp.int32) for _ in range(2)],
            [pltpu.VMEM((_IB * _K,), jnp.int32) for _ in range(2)],
            [pltpu.VMEM((_K, _D), jnp.float32) for _ in range(_NB)],
            [pltpu.SemaphoreType.DMA,
             [pltpu.SemaphoreType.DMA for _ in range(_NB)],
             [pltpu.SemaphoreType.DMA for _ in range(_NB)]],
        ],
    )
    return kern(h, src, dst)


def _dot(a, b):
    return jax.lax.dot_general(
        a, b, (((1,), (0,)), ((), ())),
        precision=jax.lax.Precision.HIGHEST,
        preferred_element_type=jnp.float32,
    )


def _row_valid():
    return jax.lax.broadcasted_iota(jnp.int32, (_NPAD, 1), 0) < _N


def _bn_relu(z, gamma, beta, rmask):
    zm = jnp.where(rmask, z, 0.0)
    mean = jnp.sum(zm, axis=0, keepdims=True) * (1.0 / _N)
    cen = z - mean
    cm = jnp.where(rmask, cen, 0.0)
    var = jnp.sum(cm * cm, axis=0, keepdims=True) * (1.0 / _N)
    return jnp.maximum(cen * jax.lax.rsqrt(var + 1e-5) * gamma + beta, 0.0)


def _mlp_body(parts, h, w1, b1, gamma, beta, w2, b2, out):
    rmask = _row_valid()
    z = parts[0] + parts[1] - h[...]
    z1 = _dot(z, w1[...]) + b1[...]
    zr = _bn_relu(z1, gamma[...], beta[...], rmask)
    val = jnp.maximum(_dot(zr, w2[...]) + b2[...], 0.0)
    out[...] = jnp.where(rmask, val, 0.0)


def _final_body(parts, h, w1, b1, gamma, beta, w2, b2, l1w, l1b, l2w, l2b, out):
    rmask = _row_valid()
    z = parts[0] + parts[1] - h[...]
    z1 = _dot(z, w1[...]) + b1[...]
    zr = _bn_relu(z1, gamma[...], beta[...], rmask)
    hh = jnp.maximum(_dot(zr, w2[...]) + b2[...], 0.0)
    hh = jnp.maximum(_dot(hh, l1w[...]) + l1b[...], 0.0)
    logits = _dot(hh, l2w[...]) + l2b[...]
    valid = jax.lax.broadcasted_iota(jnp.int32, (_NPAD, _D), 1) < _OUT
    masked = jnp.where(valid, logits, jnp.float32(-1e30))
    m = jnp.max(masked, axis=1, keepdims=True)
    lse = jnp.log(jnp.sum(jnp.where(valid, jnp.exp(masked - m), 0.0),
                          axis=1, keepdims=True))
    out[...] = logits - m - lse


def _mlp(parts, h, p):
    return pl.pallas_call(
        _mlp_body,
        out_shape=jax.ShapeDtypeStruct((_NPAD, _D), jnp.float32),
    )(parts, h, p["W1"], p["b1"].reshape(1, _D), p["gamma"].reshape(1, _D),
      p["beta"].reshape(1, _D), p["W2"], p["b2"].reshape(1, _D))


def _final(parts, h, p, l1w, l1b, l2w, l2b):
    l2w_pad = jnp.zeros((_D, _D), jnp.float32).at[:, :_OUT].set(l2w)
    l2b_pad = jnp.zeros((1, _D), jnp.float32).at[0, :_OUT].set(l2b)
    return pl.pallas_call(
        _final_body,
        out_shape=jax.ShapeDtypeStruct((_NPAD, _D), jnp.float32),
    )(parts, h, p["W1"], p["b1"].reshape(1, _D), p["gamma"].reshape(1, _D),
      p["beta"].reshape(1, _D), p["W2"], p["b2"].reshape(1, _D),
      l1w, l1b.reshape(1, _D), l2w_pad, l2b_pad)


def kernel(x, edge_index, params):
    ei = edge_index.astype(jnp.int32)
    src, dst = ei[0], ei[1]
    h = jnp.pad(x.astype(jnp.float32), ((0, _NPAD - _N), (0, 0)))
    for l in range(2):
        parts = _aggregate(h, src, dst)
        h = _mlp(parts, h, params["convs"][l])
    parts = _aggregate(h, src, dst)
    out = _final(parts, h, params["convs"][2], params["lin1_W"],
                 params["lin1_b"], params["lin2_W"], params["lin2_b"])
    return out[:_N, :_OUT]


# speedup vs baseline: 1.0076x; 1.0076x over previous
import jax
import jax.numpy as jnp
from jax.experimental import pallas as pl
from jax.experimental.pallas import tpu as pltpu
from jax.experimental.pallas import tpu_sc as plsc

_N = 10000
_NPAD = 10240
_D = 128
_E = 320000
_OUT = 47
_K = 64
_IB = 20
_NSUB = 16
_NCORES = 2
_W = _NCORES * _NSUB
_SUPERS = _E // (_IB * _K)
_NITER = -(-_SUPERS // _W)
_STRIPE = _NPAD // _NSUB

_NB = 5
_GL = 4


def _agg_body(h_hbm, src_hbm, dst_hbm, out_hbm, acc, srcs, dsts, rows, sems):
    c = jax.lax.axis_index("c")
    s = jax.lax.axis_index("s")
    wid = s * _NCORES + c
    row0 = s * _STRIPE
    sem_i, gsems, ssems = sems

    def fire_idx(sc, bank):
        base = sc * (_IB * _K)
        pltpu.async_copy(
            src_hbm.at[pl.ds(base, _IB * _K)], srcs[bank], sem_i)
        pltpu.async_copy(
            dst_hbm.at[pl.ds(base, _IB * _K)], dsts[bank], sem_i)

    def drain_idx(sc, bank):
        base = sc * (_IB * _K)
        pltpu.make_async_copy(
            src_hbm.at[pl.ds(base, _IB * _K)], srcs[bank], sem_i).wait()
        pltpu.make_async_copy(
            dst_hbm.at[pl.ds(base, _IB * _K)], dsts[bank], sem_i).wait()

    fire_idx(wid, 0)
    pltpu.async_copy(
        h_hbm.at[pl.ds(row0, _STRIPE)], acc.at[pl.ds(row0, _STRIPE)], sem_i
    ).wait()
    plsc.subcore_barrier()

    def do_super(sc, bank):
        drain_idx(sc, bank)

        nsc = sc + _W

        @pl.when(nsc < _SUPERS)
        def _():
            fire_idx(nsc, bank ^ 1)

        gd = [None] * _IB
        sd = [None] * _NB
        for j in range(min(_GL, _IB)):
            gd[j] = pltpu.async_copy(
                h_hbm.at[srcs[bank].at[pl.ds(j * _K, _K)]],
                rows[j % _NB], gsems[j % _NB])
        for j in range(_IB):
            jj = j + _GL
            if jj < _IB:
                b2 = jj % _NB
                if sd[b2] is not None:
                    sd[b2].wait()
                    sd[b2] = None
                gd[jj] = pltpu.async_copy(
                    h_hbm.at[srcs[bank].at[pl.ds(jj * _K, _K)]],
                    rows[b2], gsems[b2])
            b = j % _NB
            gd[j].wait()
            sd[b] = pltpu.async_copy(
                rows[b], acc.at[dsts[bank].at[pl.ds(j * _K, _K)]],
                ssems[b], add=True)
        for d in sd:
            if d is not None:
                d.wait()

    @pl.loop(0, _NITER, step=2)
    def _(g):
        for k in range(2):
            sc = (g + k) * _W + wid

            @pl.when(sc < _SUPERS)
            def _(sc=sc, k=k):
                do_super(sc, k)

    plsc.subcore_barrier()
    pltpu.async_copy(
        acc.at[pl.ds(row0, _STRIPE)], out_hbm.at[c, pl.ds(row0, _STRIPE)], sem_i
    ).wait()


def _aggregate(h, src, dst):
    kern = pl.kernel(
        _agg_body,
        out_type=jax.ShapeDtypeStruct((_NCORES, _NPAD, _D), jnp.float32),
        mesh=plsc.VectorSubcoreMesh(core_axis_name="c", subcore_axis_name="s"),
        scratch_types=[
            pltpu.VMEM_SHARED((_NPAD, _D), jnp.float32),
            [pltpu.VMEM((_IB * _K,), jnp.int32) for _ in range(2)],
            [pltpu.VMEM((_IB * _K,), jnp.int32) for _ in range(2)],
            [pltpu.VMEM((_K, _D), jnp.float32) for _ in range(_NB)],
            [pltpu.SemaphoreType.DMA,
             [pltpu.SemaphoreType.DMA for _ in range(_NB)],
             [pltpu.SemaphoreType.DMA for _ in range(_NB)]],
        ],
    )
    return kern(h, src, dst)


def _dot(a, b):
    return jax.lax.dot_general(
        a, b, (((1,), (0,)), ((), ())),
        precision=jax.lax.Precision.HIGHEST,
        preferred_element_type=jnp.float32,
    )


def _row_valid():
    return jax.lax.broadcasted_iota(jnp.int32, (_NPAD, 1), 0) < _N


def _bn_relu(z, gamma, beta, rmask):
    zm = jnp.where(rmask, z, 0.0)
    mean = jnp.sum(zm, axis=0, keepdims=True) * (1.0 / _N)
    cen = z - mean
    cm = jnp.where(rmask, cen, 0.0)
    var = jnp.sum(cm * cm, axis=0, keepdims=True) * (1.0 / _N)
    return jnp.maximum(cen * jax.lax.rsqrt(var + 1e-5) * gamma + beta, 0.0)


def _mlp_body(parts, h, w1, b1, gamma, beta, w2, b2, out):
    rmask = _row_valid()
    z = parts[0] + parts[1] - h[...]
    z1 = _dot(z, w1[...]) + b1[...]
    zr = _bn_relu(z1, gamma[...], beta[...], rmask)
    val = jnp.maximum(_dot(zr, w2[...]) + b2[...], 0.0)
    out[...] = jnp.where(rmask, val, 0.0)


def _final_body(parts, h, w1, b1, gamma, beta, w2, b2, l1w, l1b, l2w, l2b, out):
    rmask = _row_valid()
    z = parts[0] + parts[1] - h[...]
    z1 = _dot(z, w1[...]) + b1[...]
    zr = _bn_relu(z1, gamma[...], beta[...], rmask)
    hh = jnp.maximum(_dot(zr, w2[...]) + b2[...], 0.0)
    hh = jnp.maximum(_dot(hh, l1w[...]) + l1b[...], 0.0)
    logits = _dot(hh, l2w[...]) + l2b[...]
    valid = jax.lax.broadcasted_iota(jnp.int32, (_NPAD, _D), 1) < _OUT
    masked = jnp.where(valid, logits, jnp.float32(-1e30))
    m = jnp.max(masked, axis=1, keepdims=True)
    lse = jnp.log(jnp.sum(jnp.where(valid, jnp.exp(masked - m), 0.0),
                          axis=1, keepdims=True))
    out[...] = logits - m - lse


def _mlp(parts, h, p):
    return pl.pallas_call(
        _mlp_body,
        out_shape=jax.ShapeDtypeStruct((_NPAD, _D), jnp.float32),
    )(parts, h, p["W1"], p["b1"].reshape(1, _D), p["gamma"].reshape(1, _D),
      p["beta"].reshape(1, _D), p["W2"], p["b2"].reshape(1, _D))


def _final(parts, h, p, l1w, l1b, l2w, l2b):
    l2w_pad = jnp.zeros((_D, _D), jnp.float32).at[:, :_OUT].set(l2w)
    l2b_pad = jnp.zeros((1, _D), jnp.float32).at[0, :_OUT].set(l2b)
    return pl.pallas_call(
        _final_body,
        out_shape=jax.ShapeDtypeStruct((_NPAD, _D), jnp.float32),
    )(parts, h, p["W1"], p["b1"].reshape(1, _D), p["gamma"].reshape(1, _D),
      p["beta"].reshape(1, _D), p["W2"], p["b2"].reshape(1, _D),
      l1w, l1b.reshape(1, _D), l2w_pad, l2b_pad)


def kernel(x, edge_index, params):
    ei = edge_index.astype(jnp.int32)
    src, dst = ei[0], ei[1]
    h = jnp.pad(x.astype(jnp.float32), ((0, _NPAD - _N), (0, 0)))
    for l in range(2):
        parts = _aggregate(h, src, dst)
        h = _mlp(parts, h, params["convs"][l])
    parts = _aggregate(h, src, dst)
    out = _final(parts, h, params["convs"][2], params["lin1_W"],
                 params["lin1_b"], params["lin2_W"], params["lin2_b"])
    return out[:_N, :_OUT]
